# baseline (speedup 1.0000x reference)
import functools

import jax
import jax.numpy as jnp
from jax import lax
from jax.experimental import pallas as pl
from jax.experimental.pallas import tpu as pltpu
from jax.experimental.pallas import tpu_sc as plsc

N = 100000
NPAD = 100352
E = 3200000
CH = 128
EPAD = 3276800
ROWS = EPAD // CH
NC = 2
NS = 16
NW = NC * NS
ROWS_PER_W = ROWS // NW
RB = 10
SLICE = NPAD // NS
WCH = SLICE // 32
NCH = 32

_mesh = plsc.VectorSubcoreMesh(core_axis_name="c", subcore_axis_name="s")
_sc_params = pltpu.CompilerParams(use_tc_tiling_on_sc=False)

C0_RPT = 1070
C1_RPT = 530


def _my_rows():
    c = lax.axis_index("c")
    s = lax.axis_index("s")
    is0 = c == 0
    base = jnp.where(is0, s * C0_RPT, NS * C0_RPT + s * C1_RPT)
    niter = jnp.where(is0, C0_RPT // RB, C1_RPT // RB)
    return base, niter


@functools.partial(
    pl.kernel,
    out_type=jax.ShapeDtypeStruct((NC, NPAD), jnp.float32),
    mesh=_mesh,
    scratch_types=[
        pltpu.VMEM((RB * CH,), jnp.int32),
        pltpu.VMEM((RB * CH,), jnp.float32),
        pltpu.VMEM((SLICE,), jnp.float32),
        pltpu.VMEM_SHARED((NPAD,), jnp.float32),
        pltpu.SemaphoreType.DMA,
    ],
    compiler_params=_sc_params,
)
def _sc_deg(dst1d, ones_h, zeros_h, out, dstbuf, ones_v, stage, acc, sem):
    c = lax.axis_index("c")
    s = lax.axis_index("s")
    pltpu.sync_copy(ones_h, ones_v)
    my = pl.ds(s * SLICE, SLICE)
    pltpu.sync_copy(zeros_h.at[my], stage)
    pltpu.sync_copy(stage, acc.at[my])
    plsc.subcore_barrier()
    wbase, niter = _my_rows()

    def body(i, carry):
        e0 = (wbase + i * RB) * CH
        pltpu.sync_copy(dst1d.at[pl.ds(e0, RB * CH)], dstbuf)
        pltpu.async_copy(ones_v, acc.at[dstbuf], sem, add=True).wait()
        return carry

    lax.fori_loop(0, niter, body, 0)
    plsc.subcore_barrier()
    pltpu.sync_copy(acc.at[my], stage)
    pltpu.sync_copy(stage, out.at[c, my])


def _make_sc_agg(D):
    @functools.partial(
        pl.kernel,
        out_type=jax.ShapeDtypeStruct((NC, NPAD, D), jnp.float32),
        mesh=_mesh,
        scratch_types=[
            [pltpu.VMEM((RB * CH // 4,), jnp.int32) for _ in range(4)],
            [pltpu.VMEM((RB * CH // 4,), jnp.int32) for _ in range(4)],
            [pltpu.VMEM((RB * CH // 4, D), jnp.float32) for _ in range(4)],
            pltpu.VMEM((WCH, D), jnp.float32),
            pltpu.VMEM_SHARED((NPAD, D), jnp.float32),
            [pltpu.SemaphoreType.DMA for _ in range(4)],
            [pltpu.SemaphoreType.DMA for _ in range(4)],
        ],
        compiler_params=_sc_params,
    )
    def _sc_agg(table, src1d, dst1d, zeros_h, out,
                srcbufs, dstbufs, rows_vs, stage, acc, semg, sems):
        c = lax.axis_index("c")
        s = lax.axis_index("s")
        Q = RB * CH // 4
        for k in range(NCH):
            ch = pl.ds(s * SLICE + k * WCH, WCH)
            pltpu.sync_copy(zeros_h.at[ch, :], stage)
            pltpu.sync_copy(stage, acc.at[ch, :])
        plsc.subcore_barrier()
        wbase, niter = _my_rows()

        def drain_scatters():
            for q in range(4):
                pltpu.make_async_copy(rows_vs[q], acc.at[dstbufs[q]],
                                      sems[q]).wait()

        def body(i, carry):
            @pl.when(i > 0)
            def _():
                drain_scatters()

            e0 = (wbase + i * RB) * CH
            for q in range(4):
                pltpu.sync_copy(src1d.at[pl.ds(e0 + q * Q, Q)], srcbufs[q])
                pltpu.sync_copy(dst1d.at[pl.ds(e0 + q * Q, Q)], dstbufs[q])
            gh = [pltpu.async_copy(table.at[srcbufs[q]], rows_vs[q], semg[q])
                  for q in range(4)]
            for q in range(4):
                gh[q].wait()
                pltpu.async_copy(rows_vs[q], acc.at[dstbufs[q]], sems[q],
                                 add=True)
            return carry

        lax.fori_loop(0, niter, body, 0)
        drain_scatters()
        plsc.subcore_barrier()
        for k in range(NCH):
            ch = pl.ds(s * SLICE + k * WCH, WCH)
            pltpu.sync_copy(acc.at[ch, :], stage)
            pltpu.sync_copy(stage, out.at[c, ch, :])

    return _sc_agg


_sc_agg16 = _make_sc_agg(16)
_sc_agg8 = _make_sc_agg(8)

BLK = 6272
GRID = NPAD // BLK


def _tc_prep_body(degp_ref, x_ref, u1_ref):
    i = pl.program_id(0)
    deg = degp_ref[:, 0:1] + degp_ref[:, 1:2] + 1.0
    dinv = lax.rsqrt(deg)
    row = lax.broadcasted_iota(jnp.int32, (BLK, 1), 0) + i * BLK
    dinv = jnp.where(row < N, dinv, 0.0)
    col = lax.broadcasted_iota(jnp.int32, (BLK, 16), 1)
    u1_ref[...] = jnp.where(col == 15, dinv, dinv * x_ref[...])


def _tc_prep(degp_t, xp):
    return pl.pallas_call(
        _tc_prep_body,
        grid=(GRID,),
        in_specs=[
            pl.BlockSpec((BLK, 2), lambda i: (i, 0)),
            pl.BlockSpec((BLK, 16), lambda i: (i, 0)),
        ],
        out_specs=pl.BlockSpec((BLK, 16), lambda i: (i, 0)),
        out_shape=jax.ShapeDtypeStruct((NPAD, 16), jnp.float32),
    )(degp_t, xp)


def _tc_mid_body(s1_ref, u1_ref, W1_ref, b1_ref, W2_ref, u2_ref):
    u1 = u1_ref[...]
    dinv = u1[:, 15:16]
    t = dinv * (s1_ref[0] + s1_ref[1] + u1)
    h = jnp.dot(t, W1_ref[...], preferred_element_type=jnp.float32)
    h = jnp.maximum(h + b1_ref[...], 0.0)
    p = jnp.dot(h, W2_ref[...], preferred_element_type=jnp.float32)
    col = lax.broadcasted_iota(jnp.int32, (BLK, 8), 1)
    u2_ref[...] = jnp.where(col == 2, dinv, dinv * p)


def _tc_mid(s1p, u1, W1p, b1r, W2p):
    return pl.pallas_call(
        _tc_mid_body,
        grid=(GRID,),
        in_specs=[
            pl.BlockSpec((2, BLK, 16), lambda i: (0, i, 0)),
            pl.BlockSpec((BLK, 16), lambda i: (i, 0)),
            pl.BlockSpec((16, 35), lambda i: (0, 0)),
            pl.BlockSpec((1, 35), lambda i: (0, 0)),
            pl.BlockSpec((35, 8), lambda i: (0, 0)),
        ],
        out_specs=pl.BlockSpec((BLK, 8), lambda i: (i, 0)),
        out_shape=jax.ShapeDtypeStruct((NPAD, 8), jnp.float32),
    )(s1p, u1, W1p, b1r, W2p)


def _tc_final_body(s2_ref, u2_ref, b2_ref, out_ref):
    u2 = u2_ref[...]
    dinv = u2[:, 2:3]
    z8 = dinv * (s2_ref[0] + s2_ref[1] + u2)
    z = z8[:, 0:2] + b2_ref[...]
    m = jnp.max(z, axis=1, keepdims=True)
    e = jnp.exp(z - m)
    lse = jnp.log(jnp.sum(e, axis=1, keepdims=True)) + m
    out_ref[...] = z - lse


def _tc_final(s2p, u2, b2r):
    return pl.pallas_call(
        _tc_final_body,
        grid=(GRID,),
        in_specs=[
            pl.BlockSpec((2, BLK, 8), lambda i: (0, i, 0)),
            pl.BlockSpec((BLK, 8), lambda i: (i, 0)),
            pl.BlockSpec((1, 2), lambda i: (0, 0)),
        ],
        out_specs=pl.BlockSpec((BLK, 2), lambda i: (i, 0)),
        out_shape=jax.ShapeDtypeStruct((NPAD, 2), jnp.float32),
    )(s2p, u2, b2r)


def kernel(x, edge_index, W1, b1, W2, b2):
    pad = jnp.full((EPAD - E,), N, jnp.int32)
    src1d = jnp.concatenate([edge_index[0], pad])
    dst1d = jnp.concatenate([edge_index[1], pad])
    xp = jnp.zeros((NPAD, 16), jnp.float32).at[:N, :10].set(x)
    ones_h = jnp.ones((RB * CH,), jnp.float32)
    z1 = jnp.zeros((NPAD,), jnp.float32)
    z16 = jnp.zeros((NPAD, 16), jnp.float32)
    z8 = jnp.zeros((NPAD, 8), jnp.float32)
    W1p = jnp.zeros((16, 35), jnp.float32).at[:10].set(W1)
    W2p = jnp.zeros((35, 8), jnp.float32).at[:, :2].set(W2)
    b1r = b1.reshape(1, 35)
    b2r = b2.reshape(1, 2)

    degp = _sc_deg(dst1d, ones_h, z1)
    u1 = _tc_prep(degp.T, xp)
    s1p = _sc_agg16(u1, src1d, dst1d, z16)
    u2 = _tc_mid(s1p, u1, W1p, b1r, W2p)
    s2p = _sc_agg8(u2, src1d, dst1d, z8)
    out = _tc_final(s2p, u2, b2r)
    return out[:N]


# speedup vs baseline: 1.1880x; 1.1880x over previous
import functools

import jax
import jax.numpy as jnp
from jax import lax
from jax.experimental import pallas as pl
from jax.experimental.pallas import tpu as pltpu
from jax.experimental.pallas import tpu_sc as plsc

N = 100000
NPAD = 100352
E = 3200000
CH = 128
EPAD = 3276800
ROWS = EPAD // CH
NC = 2
NS = 16
NW = NC * NS
ROWS_PER_W = ROWS // NW
RB = 10
SLICE = NPAD // NS
WCH = SLICE // 32
NCH = 32

_mesh = plsc.VectorSubcoreMesh(core_axis_name="c", subcore_axis_name="s")
_sc_params = pltpu.CompilerParams(use_tc_tiling_on_sc=False)

C0_RPT = 1200
C1_RPT = 400


def _my_rows():
    c = lax.axis_index("c")
    s = lax.axis_index("s")
    is0 = c == 0
    base = jnp.where(is0, s * C0_RPT, NS * C0_RPT + s * C1_RPT)
    niter = jnp.where(is0, C0_RPT // RB, C1_RPT // RB)
    return base, niter


@functools.partial(
    pl.kernel,
    out_type=jax.ShapeDtypeStruct((NC, NPAD), jnp.float32),
    mesh=_mesh,
    scratch_types=[
        pltpu.VMEM((RB * CH,), jnp.int32),
        pltpu.VMEM((RB * CH,), jnp.float32),
        pltpu.VMEM((SLICE,), jnp.float32),
        pltpu.VMEM_SHARED((NPAD,), jnp.float32),
        pltpu.SemaphoreType.DMA,
    ],
    compiler_params=_sc_params,
)
def _sc_deg(dst1d, ones_h, zeros_h, out, dstbuf, ones_v, stage, acc, sem):
    c = lax.axis_index("c")
    s = lax.axis_index("s")
    pltpu.sync_copy(ones_h, ones_v)
    my = pl.ds(s * SLICE, SLICE)
    pltpu.sync_copy(zeros_h.at[my], stage)
    pltpu.sync_copy(stage, acc.at[my])
    plsc.subcore_barrier()
    wbase, niter = _my_rows()

    def body(i, carry):
        e0 = (wbase + i * RB) * CH
        pltpu.sync_copy(dst1d.at[pl.ds(e0, RB * CH)], dstbuf)
        pltpu.async_copy(ones_v, acc.at[dstbuf], sem, add=True).wait()
        return carry

    lax.fori_loop(0, niter, body, 0)
    plsc.subcore_barrier()
    pltpu.sync_copy(acc.at[my], stage)
    pltpu.sync_copy(stage, out.at[c, my])


def _make_sc_agg(D):
    @functools.partial(
        pl.kernel,
        out_type=jax.ShapeDtypeStruct((NC, NPAD, D), jnp.float32),
        mesh=_mesh,
        scratch_types=[
            pltpu.VMEM((RB * CH,), jnp.int32),
            pltpu.VMEM((RB * CH,), jnp.int32),
            pltpu.VMEM((RB * CH, D), jnp.float32),
            pltpu.VMEM((WCH, D), jnp.float32),
            pltpu.VMEM_SHARED((NPAD, D), jnp.float32),
            pltpu.SemaphoreType.DMA,
            pltpu.SemaphoreType.DMA,
        ],
        compiler_params=_sc_params,
    )
    def _sc_agg(table, src1d, dst1d, zeros_h, out,
                srcbuf, dstbuf, rows_v, stage, acc, sem, sem2):
        c = lax.axis_index("c")
        s = lax.axis_index("s")
        for k in range(NCH):
            ch = pl.ds(s * SLICE + k * WCH, WCH)
            pltpu.sync_copy(zeros_h.at[ch, :], stage)
            pltpu.sync_copy(stage, acc.at[ch, :])
        plsc.subcore_barrier()
        wbase, niter = _my_rows()

        def body(i, carry):
            e0 = (wbase + i * RB) * CH
            pltpu.sync_copy(src1d.at[pl.ds(e0, RB * CH)], srcbuf)
            pltpu.sync_copy(dst1d.at[pl.ds(e0, RB * CH)], dstbuf)
            pltpu.async_copy(table.at[srcbuf], rows_v, sem).wait()
            pltpu.async_copy(rows_v, acc.at[dstbuf], sem2, add=True).wait()
            return carry

        lax.fori_loop(0, niter, body, 0)
        plsc.subcore_barrier()
        for k in range(NCH):
            ch = pl.ds(s * SLICE + k * WCH, WCH)
            pltpu.sync_copy(acc.at[ch, :], stage)
            pltpu.sync_copy(stage, out.at[c, ch, :])

    return _sc_agg


_sc_agg16 = _make_sc_agg(16)
_sc_agg8 = _make_sc_agg(8)

BLK = 6272
GRID = NPAD // BLK


def _tc_prep_body(degp_ref, x_ref, u1_ref):
    i = pl.program_id(0)
    deg = degp_ref[:, 0:1] + degp_ref[:, 1:2] + 1.0
    dinv = lax.rsqrt(deg)
    row = lax.broadcasted_iota(jnp.int32, (BLK, 1), 0) + i * BLK
    dinv = jnp.where(row < N, dinv, 0.0)
    col = lax.broadcasted_iota(jnp.int32, (BLK, 16), 1)
    u1_ref[...] = jnp.where(col == 15, dinv, dinv * x_ref[...])


def _tc_prep(degp_t, xp):
    return pl.pallas_call(
        _tc_prep_body,
        grid=(GRID,),
        in_specs=[
            pl.BlockSpec((BLK, 2), lambda i: (i, 0)),
            pl.BlockSpec((BLK, 16), lambda i: (i, 0)),
        ],
        out_specs=pl.BlockSpec((BLK, 16), lambda i: (i, 0)),
        out_shape=jax.ShapeDtypeStruct((NPAD, 16), jnp.float32),
    )(degp_t, xp)


def _tc_mid_body(s1_ref, u1_ref, W1_ref, b1_ref, W2_ref, u2_ref):
    u1 = u1_ref[...]
    dinv = u1[:, 15:16]
    t = dinv * (s1_ref[0] + s1_ref[1] + u1)
    h = jnp.dot(t, W1_ref[...], preferred_element_type=jnp.float32)
    h = jnp.maximum(h + b1_ref[...], 0.0)
    p = jnp.dot(h, W2_ref[...], preferred_element_type=jnp.float32)
    col = lax.broadcasted_iota(jnp.int32, (BLK, 8), 1)
    u2_ref[...] = jnp.where(col == 2, dinv, dinv * p)


def _tc_mid(s1p, u1, W1p, b1r, W2p):
    return pl.pallas_call(
        _tc_mid_body,
        grid=(GRID,),
        in_specs=[
            pl.BlockSpec((2, BLK, 16), lambda i: (0, i, 0)),
            pl.BlockSpec((BLK, 16), lambda i: (i, 0)),
            pl.BlockSpec((16, 35), lambda i: (0, 0)),
            pl.BlockSpec((1, 35), lambda i: (0, 0)),
            pl.BlockSpec((35, 8), lambda i: (0, 0)),
        ],
        out_specs=pl.BlockSpec((BLK, 8), lambda i: (i, 0)),
        out_shape=jax.ShapeDtypeStruct((NPAD, 8), jnp.float32),
    )(s1p, u1, W1p, b1r, W2p)


def _tc_final_body(s2_ref, u2_ref, b2_ref, out_ref):
    u2 = u2_ref[...]
    dinv = u2[:, 2:3]
    z8 = dinv * (s2_ref[0] + s2_ref[1] + u2)
    z = z8[:, 0:2] + b2_ref[...]
    m = jnp.max(z, axis=1, keepdims=True)
    e = jnp.exp(z - m)
    lse = jnp.log(jnp.sum(e, axis=1, keepdims=True)) + m
    out_ref[...] = z - lse


def _tc_final(s2p, u2, b2r):
    return pl.pallas_call(
        _tc_final_body,
        grid=(GRID,),
        in_specs=[
            pl.BlockSpec((2, BLK, 8), lambda i: (0, i, 0)),
            pl.BlockSpec((BLK, 8), lambda i: (i, 0)),
            pl.BlockSpec((1, 2), lambda i: (0, 0)),
        ],
        out_specs=pl.BlockSpec((BLK, 2), lambda i: (i, 0)),
        out_shape=jax.ShapeDtypeStruct((NPAD, 2), jnp.float32),
    )(s2p, u2, b2r)


def kernel(x, edge_index, W1, b1, W2, b2):
    pad = jnp.full((EPAD - E,), N, jnp.int32)
    src1d = jnp.concatenate([edge_index[0], pad])
    dst1d = jnp.concatenate([edge_index[1], pad])
    xp = jnp.zeros((NPAD, 16), jnp.float32).at[:N, :10].set(x)
    ones_h = jnp.ones((RB * CH,), jnp.float32)
    z1 = jnp.zeros((NPAD,), jnp.float32)
    z16 = jnp.zeros((NPAD, 16), jnp.float32)
    z8 = jnp.zeros((NPAD, 8), jnp.float32)
    W1p = jnp.zeros((16, 35), jnp.float32).at[:10].set(W1)
    W2p = jnp.zeros((35, 8), jnp.float32).at[:, :2].set(W2)
    b1r = b1.reshape(1, 35)
    b2r = b2.reshape(1, 2)

    degp = _sc_deg(dst1d, ones_h, z1)
    u1 = _tc_prep(degp.T, xp)
    s1p = _sc_agg16(u1, src1d, dst1d, z16)
    u2 = _tc_mid(s1p, u1, W1p, b1r, W2p)
    s2p = _sc_agg8(u2, src1d, dst1d, z8)
    out = _tc_final(s2p, u2, b2r)
    return out[:N]


# speedup vs baseline: 1.2158x; 1.0234x over previous
import functools

import jax
import jax.numpy as jnp
from jax import lax
from jax.experimental import pallas as pl
from jax.experimental.pallas import tpu as pltpu
from jax.experimental.pallas import tpu_sc as plsc

N = 100000
NPAD = 100352
E = 3200000
CH = 128
EPAD = 3276800
ROWS = EPAD // CH
NC = 2
NS = 16
NW = NC * NS
ROWS_PER_W = ROWS // NW
RB = 10
SLICE = NPAD // NS
WCH = SLICE // 32
NCH = 32

_mesh = plsc.VectorSubcoreMesh(core_axis_name="c", subcore_axis_name="s")
_sc_params = pltpu.CompilerParams(use_tc_tiling_on_sc=False)

C0_RPT = 1200
C1_RPT = 400


def _my_rows():
    c = lax.axis_index("c")
    s = lax.axis_index("s")
    is0 = c == 0
    base = jnp.where(is0, s * C0_RPT, NS * C0_RPT + s * C1_RPT)
    niter = jnp.where(is0, C0_RPT // RB, C1_RPT // RB)
    return base, niter


@functools.partial(
    pl.kernel,
    out_type=jax.ShapeDtypeStruct((NC, NPAD), jnp.float32),
    mesh=_mesh,
    scratch_types=[
        pltpu.VMEM((RB * CH,), jnp.int32),
        pltpu.VMEM((RB * CH,), jnp.float32),
        pltpu.VMEM((SLICE,), jnp.float32),
        pltpu.VMEM_SHARED((NPAD,), jnp.float32),
        pltpu.SemaphoreType.DMA,
    ],
    compiler_params=_sc_params,
)
def _sc_deg(dst1d, ones_h, zeros_h, out, dstbuf, ones_v, stage, acc, sem):
    c = lax.axis_index("c")
    s = lax.axis_index("s")
    pltpu.sync_copy(ones_h, ones_v)
    my = pl.ds(s * SLICE, SLICE)
    pltpu.sync_copy(zeros_h, stage)
    pltpu.sync_copy(stage, acc.at[my])
    plsc.subcore_barrier()
    wbase, niter = _my_rows()

    def body(i, carry):
        e0 = (wbase + i * RB) * CH
        pltpu.sync_copy(dst1d.at[pl.ds(e0, RB * CH)], dstbuf)
        pltpu.async_copy(ones_v, acc.at[dstbuf], sem, add=True).wait()
        return carry

    lax.fori_loop(0, niter, body, 0)
    plsc.subcore_barrier()
    pltpu.sync_copy(acc.at[my], stage)
    pltpu.sync_copy(stage, out.at[c, my])


def _make_sc_agg(D):
    @functools.partial(
        pl.kernel,
        out_type=jax.ShapeDtypeStruct((NC, NPAD, D), jnp.float32),
        mesh=_mesh,
        scratch_types=[
            pltpu.VMEM((RB * CH,), jnp.int32),
            pltpu.VMEM((RB * CH,), jnp.int32),
            pltpu.VMEM((RB * CH, D), jnp.float32),
            pltpu.VMEM((WCH, D), jnp.float32),
            pltpu.VMEM_SHARED((NPAD, D), jnp.float32),
            pltpu.SemaphoreType.DMA,
            pltpu.SemaphoreType.DMA,
        ],
        compiler_params=_sc_params,
    )
    def _sc_agg(table, src1d, dst1d, zeros_h, out,
                srcbuf, dstbuf, rows_v, stage, acc, sem, sem2):
        c = lax.axis_index("c")
        s = lax.axis_index("s")
        pltpu.sync_copy(zeros_h, stage)
        for k in range(NCH):
            ch = pl.ds(s * SLICE + k * WCH, WCH)
            pltpu.sync_copy(stage, acc.at[ch, :])
        plsc.subcore_barrier()
        wbase, niter = _my_rows()

        def body(i, carry):
            e0 = (wbase + i * RB) * CH
            pltpu.sync_copy(src1d.at[pl.ds(e0, RB * CH)], srcbuf)
            pltpu.sync_copy(dst1d.at[pl.ds(e0, RB * CH)], dstbuf)
            pltpu.async_copy(table.at[srcbuf], rows_v, sem).wait()
            pltpu.async_copy(rows_v, acc.at[dstbuf], sem2, add=True).wait()
            return carry

        lax.fori_loop(0, niter, body, 0)
        plsc.subcore_barrier()
        for k in range(NCH):
            ch = pl.ds(s * SLICE + k * WCH, WCH)
            pltpu.sync_copy(acc.at[ch, :], stage)
            pltpu.sync_copy(stage, out.at[c, ch, :])

    return _sc_agg


_sc_agg16 = _make_sc_agg(16)
_sc_agg8 = _make_sc_agg(8)

BLK = 6272
GRID = NPAD // BLK


def _tc_prep_body(degp_ref, x_ref, u1_ref):
    i = pl.program_id(0)
    deg = degp_ref[:, 0:1] + degp_ref[:, 1:2] + 1.0
    dinv = lax.rsqrt(deg)
    row = lax.broadcasted_iota(jnp.int32, (BLK, 1), 0) + i * BLK
    dinv = jnp.where(row < N, dinv, 0.0)
    col = lax.broadcasted_iota(jnp.int32, (BLK, 16), 1)
    u1_ref[...] = jnp.where(col == 15, dinv, dinv * x_ref[...])


def _tc_prep(degp_t, xp):
    return pl.pallas_call(
        _tc_prep_body,
        grid=(GRID,),
        in_specs=[
            pl.BlockSpec((BLK, 2), lambda i: (i, 0)),
            pl.BlockSpec((BLK, 16), lambda i: (i, 0)),
        ],
        out_specs=pl.BlockSpec((BLK, 16), lambda i: (i, 0)),
        out_shape=jax.ShapeDtypeStruct((NPAD, 16), jnp.float32),
    )(degp_t, xp)


def _tc_mid_body(s1_ref, u1_ref, W1_ref, b1_ref, W2_ref, u2_ref):
    u1 = u1_ref[...]
    dinv = u1[:, 15:16]
    t = dinv * (s1_ref[0] + s1_ref[1] + u1)
    h = jnp.dot(t, W1_ref[...], preferred_element_type=jnp.float32)
    h = jnp.maximum(h + b1_ref[...], 0.0)
    p = jnp.dot(h, W2_ref[...], preferred_element_type=jnp.float32)
    col = lax.broadcasted_iota(jnp.int32, (BLK, 8), 1)
    u2_ref[...] = jnp.where(col == 2, dinv, dinv * p)


def _tc_mid(s1p, u1, W1p, b1r, W2p):
    return pl.pallas_call(
        _tc_mid_body,
        grid=(GRID,),
        in_specs=[
            pl.BlockSpec((2, BLK, 16), lambda i: (0, i, 0)),
            pl.BlockSpec((BLK, 16), lambda i: (i, 0)),
            pl.BlockSpec((16, 35), lambda i: (0, 0)),
            pl.BlockSpec((1, 35), lambda i: (0, 0)),
            pl.BlockSpec((35, 8), lambda i: (0, 0)),
        ],
        out_specs=pl.BlockSpec((BLK, 8), lambda i: (i, 0)),
        out_shape=jax.ShapeDtypeStruct((NPAD, 8), jnp.float32),
    )(s1p, u1, W1p, b1r, W2p)


def _tc_final_body(s2_ref, u2_ref, b2_ref, out_ref):
    u2 = u2_ref[...]
    dinv = u2[:, 2:3]
    z8 = dinv * (s2_ref[0] + s2_ref[1] + u2)
    z = z8[:, 0:2] + b2_ref[...]
    m = jnp.max(z, axis=1, keepdims=True)
    e = jnp.exp(z - m)
    lse = jnp.log(jnp.sum(e, axis=1, keepdims=True)) + m
    out_ref[...] = z - lse


def _tc_final(s2p, u2, b2r):
    return pl.pallas_call(
        _tc_final_body,
        grid=(GRID,),
        in_specs=[
            pl.BlockSpec((2, BLK, 8), lambda i: (0, i, 0)),
            pl.BlockSpec((BLK, 8), lambda i: (i, 0)),
            pl.BlockSpec((1, 2), lambda i: (0, 0)),
        ],
        out_specs=pl.BlockSpec((BLK, 2), lambda i: (i, 0)),
        out_shape=jax.ShapeDtypeStruct((NPAD, 2), jnp.float32),
    )(s2p, u2, b2r)


def kernel(x, edge_index, W1, b1, W2, b2):
    pad = jnp.full((EPAD - E,), N, jnp.int32)
    src1d = jnp.concatenate([edge_index[0], pad])
    dst1d = jnp.concatenate([edge_index[1], pad])
    xp = jnp.zeros((NPAD, 16), jnp.float32).at[:N, :10].set(x)
    ones_h = jnp.ones((RB * CH,), jnp.float32)
    z1 = jnp.zeros((SLICE,), jnp.float32)
    z16 = jnp.zeros((WCH, 16), jnp.float32)
    z8 = jnp.zeros((WCH, 8), jnp.float32)
    W1p = jnp.zeros((16, 35), jnp.float32).at[:10].set(W1)
    W2p = jnp.zeros((35, 8), jnp.float32).at[:, :2].set(W2)
    b1r = b1.reshape(1, 35)
    b2r = b2.reshape(1, 2)

    degp = _sc_deg(dst1d, ones_h, z1)
    u1 = _tc_prep(degp.T, xp)
    s1p = _sc_agg16(u1, src1d, dst1d, z16)
    u2 = _tc_mid(s1p, u1, W1p, b1r, W2p)
    s2p = _sc_agg8(u2, src1d, dst1d, z8)
    out = _tc_final(s2p, u2, b2r)
    return out[:N]


# speedup vs baseline: 1.2431x; 1.0225x over previous
import functools

import jax
import jax.numpy as jnp
from jax import lax
from jax.experimental import pallas as pl
from jax.experimental.pallas import tpu as pltpu
from jax.experimental.pallas import tpu_sc as plsc

N = 100000
NPAD = 100352
E = 3200000
CH = 128
EPAD = 3276800
ROWS = EPAD // CH
NC = 2
NS = 16
NW = NC * NS
ROWS_PER_W = ROWS // NW
RB = 10
SLICE = NPAD // NS
WCH = SLICE // 32
NCH = 32

_mesh = plsc.VectorSubcoreMesh(core_axis_name="c", subcore_axis_name="s")
_sc_params = pltpu.CompilerParams(use_tc_tiling_on_sc=False)

C0_RPT = 1200
C1_RPT = 400


def _my_rows():
    c = lax.axis_index("c")
    s = lax.axis_index("s")
    is0 = c == 0
    base = jnp.where(is0, s * C0_RPT, NS * C0_RPT + s * C1_RPT)
    niter = jnp.where(is0, C0_RPT // RB, C1_RPT // RB)
    return base, niter


@functools.partial(
    pl.kernel,
    out_type=jax.ShapeDtypeStruct((NC, NPAD), jnp.float32),
    mesh=_mesh,
    scratch_types=[
        pltpu.VMEM((RB * CH,), jnp.int32),
        pltpu.VMEM((RB * CH,), jnp.float32),
        pltpu.VMEM((SLICE,), jnp.float32),
        pltpu.VMEM_SHARED((NPAD,), jnp.float32),
        pltpu.SemaphoreType.DMA,
    ],
    compiler_params=_sc_params,
)
def _sc_deg(dst1d, ones_h, zeros_h, out, dstbuf, ones_v, stage, acc, sem):
    c = lax.axis_index("c")
    s = lax.axis_index("s")
    pltpu.sync_copy(ones_h, ones_v)
    my = pl.ds(s * SLICE, SLICE)
    pltpu.sync_copy(zeros_h, stage)
    pltpu.sync_copy(stage, acc.at[my])
    plsc.subcore_barrier()
    wbase, niter = _my_rows()

    def body(i, carry):
        e0 = (wbase + i * RB) * CH
        pltpu.sync_copy(dst1d.at[pl.ds(e0, RB * CH)], dstbuf)
        pltpu.async_copy(ones_v, acc.at[dstbuf], sem, add=True).wait()
        return carry

    lax.fori_loop(0, niter, body, 0)
    plsc.subcore_barrier()
    pltpu.sync_copy(acc.at[my], stage)
    pltpu.sync_copy(stage, out.at[c, my])


def _make_sc_agg(D):
    @functools.partial(
        pl.kernel,
        out_type=jax.ShapeDtypeStruct((NC, NPAD, D), jnp.float32),
        mesh=_mesh,
        scratch_types=[
            pltpu.VMEM((RB * CH,), jnp.int32),
            pltpu.VMEM((RB * CH,), jnp.int32),
            pltpu.VMEM((RB * CH, D), jnp.float32),
            pltpu.VMEM_SHARED((NPAD, D), jnp.float32),
            pltpu.SemaphoreType.DMA,
            pltpu.SemaphoreType.DMA,
        ],
        compiler_params=_sc_params,
    )
    def _sc_agg(table, src1d, dst1d, zeros_h, out,
                srcbuf, dstbuf, rows_v, acc, sem, sem2):
        c = lax.axis_index("c")
        s = lax.axis_index("s")
        B = RB * CH
        TAIL = SLICE - 4 * B
        pltpu.sync_copy(zeros_h, rows_v)
        for k in range(4):
            pltpu.sync_copy(rows_v, acc.at[pl.ds(s * SLICE + k * B, B), :])
        pltpu.sync_copy(rows_v.at[pl.ds(0, TAIL), :],
                        acc.at[pl.ds(s * SLICE + 4 * B, TAIL), :])
        plsc.subcore_barrier()
        wbase, niter = _my_rows()

        def body(i, carry):
            e0 = (wbase + i * RB) * CH
            pltpu.sync_copy(src1d.at[pl.ds(e0, RB * CH)], srcbuf)
            pltpu.sync_copy(dst1d.at[pl.ds(e0, RB * CH)], dstbuf)
            pltpu.async_copy(table.at[srcbuf], rows_v, sem).wait()
            pltpu.async_copy(rows_v, acc.at[dstbuf], sem2, add=True).wait()
            return carry

        lax.fori_loop(0, niter, body, 0)
        plsc.subcore_barrier()
        for k in range(4):
            ch = pl.ds(s * SLICE + k * B, B)
            pltpu.sync_copy(acc.at[ch, :], rows_v)
            pltpu.sync_copy(rows_v, out.at[c, ch, :])
        cht = pl.ds(s * SLICE + 4 * B, TAIL)
        pltpu.sync_copy(acc.at[cht, :], rows_v.at[pl.ds(0, TAIL), :])
        pltpu.sync_copy(rows_v.at[pl.ds(0, TAIL), :], out.at[c, cht, :])

    return _sc_agg


_sc_agg16 = _make_sc_agg(16)
_sc_agg8 = _make_sc_agg(8)

BLK = 6272
GRID = NPAD // BLK


def _tc_prep_body(degp_ref, x_ref, u1_ref):
    i = pl.program_id(0)
    deg = degp_ref[:, 0:1] + degp_ref[:, 1:2] + 1.0
    dinv = lax.rsqrt(deg)
    row = lax.broadcasted_iota(jnp.int32, (BLK, 1), 0) + i * BLK
    dinv = jnp.where(row < N, dinv, 0.0)
    col = lax.broadcasted_iota(jnp.int32, (BLK, 16), 1)
    u1_ref[...] = jnp.where(col == 15, dinv, dinv * x_ref[...])


def _tc_prep(degp_t, xp):
    return pl.pallas_call(
        _tc_prep_body,
        grid=(GRID,),
        in_specs=[
            pl.BlockSpec((BLK, 2), lambda i: (i, 0)),
            pl.BlockSpec((BLK, 16), lambda i: (i, 0)),
        ],
        out_specs=pl.BlockSpec((BLK, 16), lambda i: (i, 0)),
        out_shape=jax.ShapeDtypeStruct((NPAD, 16), jnp.float32),
    )(degp_t, xp)


def _tc_mid_body(s1_ref, u1_ref, W1_ref, b1_ref, W2_ref, u2_ref):
    u1 = u1_ref[...]
    dinv = u1[:, 15:16]
    t = dinv * (s1_ref[0] + s1_ref[1] + u1)
    h = jnp.dot(t, W1_ref[...], preferred_element_type=jnp.float32)
    h = jnp.maximum(h + b1_ref[...], 0.0)
    p = jnp.dot(h, W2_ref[...], preferred_element_type=jnp.float32)
    col = lax.broadcasted_iota(jnp.int32, (BLK, 8), 1)
    u2_ref[...] = jnp.where(col == 2, dinv, dinv * p)


def _tc_mid(s1p, u1, W1p, b1r, W2p):
    return pl.pallas_call(
        _tc_mid_body,
        grid=(GRID,),
        in_specs=[
            pl.BlockSpec((2, BLK, 16), lambda i: (0, i, 0)),
            pl.BlockSpec((BLK, 16), lambda i: (i, 0)),
            pl.BlockSpec((16, 35), lambda i: (0, 0)),
            pl.BlockSpec((1, 35), lambda i: (0, 0)),
            pl.BlockSpec((35, 8), lambda i: (0, 0)),
        ],
        out_specs=pl.BlockSpec((BLK, 8), lambda i: (i, 0)),
        out_shape=jax.ShapeDtypeStruct((NPAD, 8), jnp.float32),
    )(s1p, u1, W1p, b1r, W2p)


BLK2 = 5000
GRID2 = N // BLK2


def _tc_final_body(s2_ref, u2_ref, b2_ref, out_ref):
    u2 = u2_ref[...]
    dinv = u2[:, 2:3]
    z8 = dinv * (s2_ref[0] + s2_ref[1] + u2)
    z = z8[:, 0:2] + b2_ref[...]
    m = jnp.max(z, axis=1, keepdims=True)
    e = jnp.exp(z - m)
    lse = jnp.log(jnp.sum(e, axis=1, keepdims=True)) + m
    out_ref[...] = z - lse


def _tc_final(s2p, u2, b2r):
    return pl.pallas_call(
        _tc_final_body,
        grid=(GRID2,),
        in_specs=[
            pl.BlockSpec((2, BLK2, 8), lambda i: (0, i, 0)),
            pl.BlockSpec((BLK2, 8), lambda i: (i, 0)),
            pl.BlockSpec((1, 2), lambda i: (0, 0)),
        ],
        out_specs=pl.BlockSpec((BLK2, 2), lambda i: (i, 0)),
        out_shape=jax.ShapeDtypeStruct((N, 2), jnp.float32),
    )(s2p, u2, b2r)


def kernel(x, edge_index, W1, b1, W2, b2):
    pad = jnp.full((EPAD - E,), N, jnp.int32)
    src1d = jnp.concatenate([edge_index[0], pad])
    dst1d = jnp.concatenate([edge_index[1], pad])
    xp = jnp.zeros((NPAD, 16), jnp.float32).at[:N, :10].set(x)
    ones_h = jnp.ones((RB * CH,), jnp.float32)
    z1 = jnp.zeros((SLICE,), jnp.float32)
    z16 = jnp.zeros((RB * CH, 16), jnp.float32)
    z8 = jnp.zeros((RB * CH, 8), jnp.float32)
    W1p = jnp.zeros((16, 35), jnp.float32).at[:10].set(W1)
    W2p = jnp.zeros((35, 8), jnp.float32).at[:, :2].set(W2)
    b1r = b1.reshape(1, 35)
    b2r = b2.reshape(1, 2)

    degp = _sc_deg(dst1d, ones_h, z1)
    u1 = _tc_prep(degp.T, xp)
    s1p = _sc_agg16(u1, src1d, dst1d, z16)
    u2 = _tc_mid(s1p, u1, W1p, b1r, W2p)
    s2p = _sc_agg8(u2, src1d, dst1d, z8)
    return _tc_final(s2p, u2, b2r)
